# baseline (speedup 1.0000x reference)
import functools

import jax
import jax.numpy as jnp
from jax import lax
from jax.experimental import pallas as pl
from jax.experimental.pallas import tpu as pltpu
from jax.experimental.pallas import tpu_sc as plsc

E = 8
TOP_K = 2
BLK = 256
NC, NS = 2, 16
NW = NC * NS


def _router_body(x_ref, wrt_ref, i1_ref, i2_ref, w1_ref, w2_ref,
                 r1_ref, r2_ref, cnt_ref, tri_ref, carry_ref):
    TB = x_ref.shape[0]
    pid = pl.program_id(0)

    @pl.when(pid == 0)
    def _init():
        r = lax.broadcasted_iota(jnp.int32, (TB, TB), 0)
        c = lax.broadcasted_iota(jnp.int32, (TB, TB), 1)
        tri_ref[...] = jnp.where(r > c, 1.0, 0.0)
        carry_ref[...] = jnp.zeros_like(carry_ref)

    x = x_ref[...]
    logits = jnp.dot(x, wrt_ref[...], preferred_element_type=jnp.float32)
    iota = lax.broadcasted_iota(jnp.int32, logits.shape, 1)
    m1 = jnp.max(logits, axis=1, keepdims=True)
    i1 = jnp.min(jnp.where(logits == m1, iota, E), axis=1, keepdims=True)
    l2 = jnp.where(iota == i1, jnp.float32(-3.0e38), logits)
    m2 = jnp.max(l2, axis=1, keepdims=True)
    i2 = jnp.min(jnp.where(l2 == m2, iota, E), axis=1, keepdims=True)
    e2 = jnp.exp(m2 - m1)
    den = 1.0 + e2
    i1_ref[...] = i1
    i2_ref[...] = i2
    w1_ref[...] = 1.0 / den
    w2_ref[...] = e2 / den

    ioe = lax.broadcasted_iota(jnp.int32, (TB, E), 1)
    oh1 = jnp.where(ioe == i1, 1.0, 0.0)
    oh2 = jnp.where(ioe == i2, 1.0, 0.0)
    tot = oh1 + oh2
    excl = jnp.dot(tri_ref[...], tot, preferred_element_type=jnp.float32)
    c1 = excl + carry_ref[...]
    r1_ref[...] = jnp.sum(c1 * oh1, axis=1, keepdims=True).astype(jnp.int32)
    r2_ref[...] = jnp.sum(c1 * oh2, axis=1, keepdims=True).astype(jnp.int32)
    new_carry = carry_ref[...] + jnp.sum(tot, axis=0, keepdims=True)
    carry_ref[...] = new_carry
    cnt_ref[...] = new_carry.astype(jnp.int32)


def _router(xf, W_router):
    N, D = xf.shape
    TB = 1024
    grid = (N // TB,)
    out_shapes = (
        jax.ShapeDtypeStruct((N, 1), jnp.int32),
        jax.ShapeDtypeStruct((N, 1), jnp.int32),
        jax.ShapeDtypeStruct((N, 1), jnp.float32),
        jax.ShapeDtypeStruct((N, 1), jnp.float32),
        jax.ShapeDtypeStruct((N, 1), jnp.int32),
        jax.ShapeDtypeStruct((N, 1), jnp.int32),
        jax.ShapeDtypeStruct((1, E), jnp.int32),
    )
    spec1 = pl.BlockSpec((TB, 1), lambda i: (i, 0))
    cspec = pl.BlockSpec((1, E), lambda i: (0, 0))
    return pl.pallas_call(
        _router_body,
        grid=grid,
        in_specs=[
            pl.BlockSpec((TB, D), lambda i: (i, 0)),
            pl.BlockSpec((D, E), lambda i: (0, 0)),
        ],
        out_specs=(spec1, spec1, spec1, spec1, spec1, spec1, cspec),
        out_shape=out_shapes,
        scratch_shapes=[
            pltpu.VMEM((TB, TB), jnp.float32),
            pltpu.VMEM((1, E), jnp.float32),
        ],
    )(xf, W_router.T)


def _sc_gather(table, idx, rows_per_w, chunk):
    P = idx.shape[0]
    D = table.shape[1]
    dt = table.dtype
    nch = rows_per_w // chunk
    mesh = plsc.VectorSubcoreMesh(core_axis_name="c", subcore_axis_name="s",
                                  num_cores=NC, num_subcores=NS)

    half = chunk // 2

    @functools.partial(
        pl.kernel,
        out_type=jax.ShapeDtypeStruct((P, D), dt),
        mesh=mesh,
        scratch_types=[
            pltpu.VMEM((rows_per_w,), jnp.int32),
            pltpu.VMEM((half, D), dt),
            pltpu.VMEM((half, D), dt),
            pltpu.SemaphoreType.DMA,
            pltpu.SemaphoreType.DMA,
        ],
    )
    def k(table_hbm, idx_hbm, out_hbm, idx_v, r0_v, r1_v, s0, s1):
        wid = lax.axis_index("s") * NC + lax.axis_index("c")
        base = wid * rows_per_w
        pltpu.sync_copy(idx_hbm.at[pl.ds(base, rows_per_w)], idx_v)

        def body(j, carry):
            o = j * chunk
            c0 = pltpu.async_copy(
                table_hbm.at[idx_v.at[pl.ds(o, half)]], r0_v, s0)
            c1 = pltpu.async_copy(
                table_hbm.at[idx_v.at[pl.ds(o + half, half)]], r1_v, s1)
            c0.wait()
            pltpu.sync_copy(r0_v, out_hbm.at[pl.ds(base + o, half)])
            c1.wait()
            pltpu.sync_copy(r1_v, out_hbm.at[pl.ds(base + o + half, half)])
            return carry

        lax.fori_loop(0, nch, body, 0)

    return k(table, idx)


def _sc_combine(table, idx_a, idx_b, rows_per_w, chunk):
    N = idx_a.shape[0]
    D = table.shape[1]
    nch = rows_per_w // chunk
    nvec = D // 16
    mesh = plsc.VectorSubcoreMesh(core_axis_name="c", subcore_axis_name="s",
                                  num_cores=NC, num_subcores=NS)

    @functools.partial(
        pl.kernel,
        out_type=jax.ShapeDtypeStruct((N, D), jnp.float32),
        mesh=mesh,
        scratch_types=[
            pltpu.VMEM((chunk,), jnp.int32),
            pltpu.VMEM((chunk,), jnp.int32),
            pltpu.VMEM((chunk, D), jnp.float32),
            pltpu.VMEM((chunk, D), jnp.float32),
            pltpu.SemaphoreType.DMA,
            pltpu.SemaphoreType.DMA,
        ],
    )
    def k(table_hbm, ia_hbm, ib_hbm, out_hbm, ia_v, ib_v, a_v, b_v, sa, sb):
        wid = lax.axis_index("s") * NC + lax.axis_index("c")
        base = wid * rows_per_w

        def body(j, carry):
            b0 = base + j * chunk
            pltpu.sync_copy(ia_hbm.at[pl.ds(b0, chunk)], ia_v)
            pltpu.sync_copy(ib_hbm.at[pl.ds(b0, chunk)], ib_v)
            ca = pltpu.async_copy(table_hbm.at[ia_v], a_v, sa)
            cb = pltpu.async_copy(table_hbm.at[ib_v], b_v, sb)
            ca.wait()
            cb.wait()

            def row(r, carry2):
                for v in range(nvec):
                    sl = pl.ds(v * 16, 16)
                    a_v[r, sl] = a_v[r, sl] + b_v[r, sl]
                return carry2

            lax.fori_loop(0, chunk, row, 0)
            pltpu.sync_copy(a_v, out_hbm.at[pl.ds(b0, chunk)])
            return carry

        lax.fori_loop(0, nch, body, 0)

    return k(table, idx_a, idx_b)


def _ffn_body(be_ref, xp_ref, w1_ref, w2_ref, wp_ref, out_ref):
    e = be_ref[pl.program_id(0)]
    x = xp_ref[...].astype(jnp.bfloat16)
    h = jnp.dot(x, w1_ref[e], preferred_element_type=jnp.float32)
    h = 0.5 * h * (1.0 + lax.erf(h * 0.7071067811865476))
    o = jnp.dot(h.astype(jnp.bfloat16), w2_ref[e],
                preferred_element_type=jnp.float32)
    out_ref[...] = o * wp_ref[...]


def _grouped_ffn(block_expert, xp, W1, W2, wp):
    P, D = xp.shape
    FF = W1.shape[2]
    nb = P // BLK
    grid_spec = pltpu.PrefetchScalarGridSpec(
        num_scalar_prefetch=1,
        grid=(nb,),
        in_specs=[
            pl.BlockSpec((BLK, D), lambda i, be: (i, 0)),
            pl.BlockSpec((E, D, FF), lambda i, be: (0, 0, 0)),
            pl.BlockSpec((E, FF, D), lambda i, be: (0, 0, 0)),
            pl.BlockSpec((BLK, 1), lambda i, be: (i, 0)),
        ],
        out_specs=pl.BlockSpec((BLK, D), lambda i, be: (i, 0)),
    )
    return pl.pallas_call(
        _ffn_body,
        grid_spec=grid_spec,
        out_shape=jax.ShapeDtypeStruct((P, D), jnp.float32),
    )(block_expert, xp, W1, W2, wp)


def kernel(x, W_router, W1, W2):
    B, T, D = x.shape
    N = B * T
    xf = x.reshape(N, D)

    i1, i2, w1, w2, r1, r2, counts = _router(xf, W_router)
    counts = counts.reshape(E)

    e_pairs = jnp.concatenate([i1, i2], axis=1).reshape(-1)
    w_pairs = jnp.concatenate([w1, w2], axis=1).reshape(-1)
    rank = jnp.concatenate([r1, r2], axis=1).reshape(-1)
    padded = ((counts + BLK - 1) // BLK) * BLK
    starts = jnp.concatenate(
        [jnp.zeros((1,), jnp.int32), jnp.cumsum(padded)[:-1].astype(jnp.int32)])
    dst = starts[e_pairs] + rank

    P = N * TOP_K + E * BLK
    nb = P // BLK
    src_tok = (jnp.arange(N * TOP_K, dtype=jnp.int32) // TOP_K)
    rev = jnp.zeros((P,), jnp.int32).at[dst].set(src_tok)
    wp = jnp.zeros((P,), jnp.float32).at[dst].set(w_pairs)
    ends = (starts + padded).astype(jnp.int32)
    bstart = jnp.arange(nb, dtype=jnp.int32) * BLK
    block_expert = jnp.minimum(
        jnp.sum((bstart[:, None] >= ends[None, :]).astype(jnp.int32), axis=1),
        E - 1).astype(jnp.int32)

    xp = _sc_gather(xf, rev, rows_per_w=P // NW, chunk=64)
    op = _grouped_ffn(block_expert, xp, W1.astype(jnp.bfloat16),
                      W2.astype(jnp.bfloat16), wp.reshape(P, 1))
    dst2 = dst.reshape(N, TOP_K)
    out = _sc_combine(op, dst2[:, 0], dst2[:, 1],
                      rows_per_w=N // NW, chunk=32)
    return out.reshape(B, T, D)


# speedup vs baseline: 1.2282x; 1.2282x over previous
import functools

import jax
import jax.numpy as jnp
from jax import lax
from jax.experimental import pallas as pl
from jax.experimental.pallas import tpu as pltpu
from jax.experimental.pallas import tpu_sc as plsc

E = 8
TOP_K = 2
BLK = 256
NC, NS = 2, 16
NW = NC * NS


def _router_body(x_ref, wrt_ref, i1_ref, i2_ref, w1_ref, w2_ref,
                 r1_ref, r2_ref, cnt_ref, tri_ref, carry_ref):
    TB = x_ref.shape[0]
    pid = pl.program_id(0)

    @pl.when(pid == 0)
    def _init():
        r = lax.broadcasted_iota(jnp.int32, (TB, TB), 0)
        c = lax.broadcasted_iota(jnp.int32, (TB, TB), 1)
        tri_ref[...] = jnp.where(r > c, 1.0, 0.0)
        carry_ref[...] = jnp.zeros_like(carry_ref)

    x = x_ref[...]
    logits = jnp.dot(x, wrt_ref[...], preferred_element_type=jnp.float32)
    iota = lax.broadcasted_iota(jnp.int32, logits.shape, 1)
    m1 = jnp.max(logits, axis=1, keepdims=True)
    i1 = jnp.min(jnp.where(logits == m1, iota, E), axis=1, keepdims=True)
    l2 = jnp.where(iota == i1, jnp.float32(-3.0e38), logits)
    m2 = jnp.max(l2, axis=1, keepdims=True)
    i2 = jnp.min(jnp.where(l2 == m2, iota, E), axis=1, keepdims=True)
    e2 = jnp.exp(m2 - m1)
    den = 1.0 + e2
    i1_ref[...] = i1
    i2_ref[...] = i2
    w1_ref[...] = 1.0 / den
    w2_ref[...] = e2 / den

    ioe = lax.broadcasted_iota(jnp.int32, (TB, 2 * E), 1)
    oh1 = jnp.where(ioe == i1, 1.0, 0.0)
    oh2 = jnp.where(ioe == i2, 1.0, 0.0)
    tot = oh1 + oh2
    excl = jnp.dot(tri_ref[...], tot, preferred_element_type=jnp.float32)
    c1 = excl + carry_ref[...]
    r1_ref[...] = jnp.sum(c1 * oh1, axis=1, keepdims=True).astype(jnp.int32)
    r2_ref[...] = jnp.sum(c1 * oh2, axis=1, keepdims=True).astype(jnp.int32)
    new_carry = carry_ref[...] + jnp.sum(tot, axis=0, keepdims=True)
    carry_ref[...] = new_carry
    cnt_ref[...] = new_carry.astype(jnp.int32)


def _router(xf, W_router):
    N, D = xf.shape
    TB = 1024
    grid = (N // TB,)
    out_shapes = (
        jax.ShapeDtypeStruct((N, 1), jnp.int32),
        jax.ShapeDtypeStruct((N, 1), jnp.int32),
        jax.ShapeDtypeStruct((N, 1), jnp.float32),
        jax.ShapeDtypeStruct((N, 1), jnp.float32),
        jax.ShapeDtypeStruct((N, 1), jnp.int32),
        jax.ShapeDtypeStruct((N, 1), jnp.int32),
        jax.ShapeDtypeStruct((1, 2 * E), jnp.int32),
    )
    spec1 = pl.BlockSpec((TB, 1), lambda i: (i, 0))
    cspec = pl.BlockSpec((1, 2 * E), lambda i: (0, 0))
    return pl.pallas_call(
        _router_body,
        grid=grid,
        in_specs=[
            pl.BlockSpec((TB, D), lambda i: (i, 0)),
            pl.BlockSpec((D, E), lambda i: (0, 0)),
        ],
        out_specs=(spec1, spec1, spec1, spec1, spec1, spec1, cspec),
        out_shape=out_shapes,
        scratch_shapes=[
            pltpu.VMEM((TB, TB), jnp.float32),
            pltpu.VMEM((1, 2 * E), jnp.float32),
        ],
    )(xf, W_router.T)


def _sc_finalize(i1, i2, r1, r2, w1, w2, counts16, P):
    N = i1.shape[0]
    ng = N // 16
    mesh = plsc.VectorSubcoreMesh(core_axis_name="c", subcore_axis_name="s",
                                  num_cores=NC, num_subcores=NS)

    @functools.partial(
        pl.kernel,
        out_type=(
            jax.ShapeDtypeStruct((P,), jnp.int32),
            jax.ShapeDtypeStruct((P,), jnp.float32),
            jax.ShapeDtypeStruct((N,), jnp.int32),
            jax.ShapeDtypeStruct((N,), jnp.int32),
        ),
        mesh=mesh,
        scratch_types=[
            pltpu.VMEM((N,), jnp.int32),
            pltpu.VMEM((N,), jnp.int32),
            pltpu.VMEM((N,), jnp.int32),
            pltpu.VMEM((N,), jnp.int32),
            pltpu.VMEM((N,), jnp.float32),
            pltpu.VMEM((N,), jnp.float32),
            pltpu.VMEM((16,), jnp.int32),
            pltpu.VMEM((P,), jnp.int32),
            pltpu.VMEM((P,), jnp.float32),
            pltpu.VMEM((N,), jnp.int32),
            pltpu.VMEM((N,), jnp.int32),
        ],
        compiler_params=pltpu.CompilerParams(needs_layout_passes=False),
    )
    def k(i1_h, i2_h, r1_h, r2_h, w1_h, w2_h, c_h,
          rev_h, wp_h, dA_h, dB_h,
          i1_v, i2_v, r1_v, r2_v, w1_v, w2_v, st_v,
          rev_v, wp_v, dA_v, dB_v):
        wid = lax.axis_index("s") * NC + lax.axis_index("c")

        @pl.when(wid == 0)
        def _work():
            pltpu.sync_copy(i1_h, i1_v)
            pltpu.sync_copy(i2_h, i2_v)
            pltpu.sync_copy(r1_h, r1_v)
            pltpu.sync_copy(r2_h, r2_v)
            pltpu.sync_copy(w1_h, w1_v)
            pltpu.sync_copy(w2_h, w2_v)
            pltpu.sync_copy(c_h, st_v)
            c16 = st_v[...]
            padded = ((c16 + 255) >> 8) << 8
            cum = plsc.cumsum(padded)
            st_v[...] = cum - padded

            zero16 = jnp.zeros((16,), jnp.int32)
            zf16 = jnp.zeros((16,), jnp.float32)

            def zinit(g, carry):
                rev_v[pl.ds(g * 16, 16)] = zero16
                wp_v[pl.ds(g * 16, 16)] = zf16
                return carry

            lax.fori_loop(0, P // 16, zinit, 0)

            lane = lax.broadcasted_iota(jnp.int32, (16,), 0)

            def body(g, carry):
                sl = pl.ds(g * 16, 16)
                tok = lane + g * 16
                e1 = i1_v[sl]
                e2 = i2_v[sl]
                dA = plsc.load_gather(st_v, [e1]) + r1_v[sl]
                dB = plsc.load_gather(st_v, [e2]) + r2_v[sl]
                dA_v[sl] = dA
                dB_v[sl] = dB
                plsc.store_scatter(rev_v, [dA], tok)
                plsc.store_scatter(rev_v, [dB], tok)
                plsc.store_scatter(wp_v, [dA], w1_v[sl])
                plsc.store_scatter(wp_v, [dB], w2_v[sl])
                return carry

            lax.fori_loop(0, ng, body, 0)
            pltpu.sync_copy(rev_v, rev_h)
            pltpu.sync_copy(wp_v, wp_h)
            pltpu.sync_copy(dA_v, dA_h)
            pltpu.sync_copy(dB_v, dB_h)

    return k(i1, i2, r1, r2, w1, w2, counts16)


def _sc_gather(table, idx, rows_per_w, chunk):
    P = idx.shape[0]
    D = table.shape[1]
    dt = table.dtype
    nch = rows_per_w // chunk
    mesh = plsc.VectorSubcoreMesh(core_axis_name="c", subcore_axis_name="s",
                                  num_cores=NC, num_subcores=NS)

    half = chunk // 2

    @functools.partial(
        pl.kernel,
        out_type=jax.ShapeDtypeStruct((P, D), dt),
        mesh=mesh,
        scratch_types=[
            pltpu.VMEM((rows_per_w,), jnp.int32),
            pltpu.VMEM((half, D), dt),
            pltpu.VMEM((half, D), dt),
            pltpu.SemaphoreType.DMA,
            pltpu.SemaphoreType.DMA,
        ],
    )
    def k(table_hbm, idx_hbm, out_hbm, idx_v, r0_v, r1_v, s0, s1):
        wid = lax.axis_index("s") * NC + lax.axis_index("c")
        base = wid * rows_per_w
        pltpu.sync_copy(idx_hbm.at[pl.ds(base, rows_per_w)], idx_v)

        def body(j, carry):
            o = j * chunk
            c0 = pltpu.async_copy(
                table_hbm.at[idx_v.at[pl.ds(o, half)]], r0_v, s0)
            c1 = pltpu.async_copy(
                table_hbm.at[idx_v.at[pl.ds(o + half, half)]], r1_v, s1)
            c0.wait()
            pltpu.sync_copy(r0_v, out_hbm.at[pl.ds(base + o, half)])
            c1.wait()
            pltpu.sync_copy(r1_v, out_hbm.at[pl.ds(base + o + half, half)])
            return carry

        lax.fori_loop(0, nch, body, 0)

    return k(table, idx)


def _sc_combine(table, idx_a, idx_b, rows_per_w, chunk):
    N = idx_a.shape[0]
    D = table.shape[1]
    nch = rows_per_w // chunk
    nvec = D // 16
    mesh = plsc.VectorSubcoreMesh(core_axis_name="c", subcore_axis_name="s",
                                  num_cores=NC, num_subcores=NS)

    @functools.partial(
        pl.kernel,
        out_type=jax.ShapeDtypeStruct((N, D), jnp.float32),
        mesh=mesh,
        scratch_types=[
            pltpu.VMEM((chunk,), jnp.int32),
            pltpu.VMEM((chunk,), jnp.int32),
            pltpu.VMEM((chunk, D), jnp.float32),
            pltpu.VMEM((chunk, D), jnp.float32),
            pltpu.SemaphoreType.DMA,
            pltpu.SemaphoreType.DMA,
        ],
    )
    def k(table_hbm, ia_hbm, ib_hbm, out_hbm, ia_v, ib_v, a_v, b_v, sa, sb):
        wid = lax.axis_index("s") * NC + lax.axis_index("c")
        base = wid * rows_per_w

        def body(j, carry):
            b0 = base + j * chunk
            pltpu.sync_copy(ia_hbm.at[pl.ds(b0, chunk)], ia_v)
            pltpu.sync_copy(ib_hbm.at[pl.ds(b0, chunk)], ib_v)
            ca = pltpu.async_copy(table_hbm.at[ia_v], a_v, sa)
            cb = pltpu.async_copy(table_hbm.at[ib_v], b_v, sb)
            ca.wait()
            cb.wait()

            def row(r, carry2):
                for v in range(nvec):
                    sl = pl.ds(v * 16, 16)
                    a_v[r, sl] = a_v[r, sl] + b_v[r, sl]
                return carry2

            lax.fori_loop(0, chunk, row, 0)
            pltpu.sync_copy(a_v, out_hbm.at[pl.ds(b0, chunk)])
            return carry

        lax.fori_loop(0, nch, body, 0)

    return k(table, idx_a, idx_b)


def _ffn_body(be_ref, xp_ref, w1_ref, w2_ref, wp_ref, out_ref):
    e = be_ref[pl.program_id(0)]
    x = xp_ref[...].astype(jnp.bfloat16)
    h = jnp.dot(x, w1_ref[e], preferred_element_type=jnp.float32)
    h = 0.5 * h * (1.0 + lax.erf(h * 0.7071067811865476))
    o = jnp.dot(h.astype(jnp.bfloat16), w2_ref[e],
                preferred_element_type=jnp.float32)
    out_ref[...] = o * wp_ref[...]


def _grouped_ffn(block_expert, xp, W1, W2, wp):
    P, D = xp.shape
    FF = W1.shape[2]
    nb = P // BLK
    grid_spec = pltpu.PrefetchScalarGridSpec(
        num_scalar_prefetch=1,
        grid=(nb,),
        in_specs=[
            pl.BlockSpec((BLK, D), lambda i, be: (i, 0)),
            pl.BlockSpec((E, D, FF), lambda i, be: (0, 0, 0)),
            pl.BlockSpec((E, FF, D), lambda i, be: (0, 0, 0)),
            pl.BlockSpec((BLK, 1), lambda i, be: (i, 0)),
        ],
        out_specs=pl.BlockSpec((BLK, D), lambda i, be: (i, 0)),
    )
    return pl.pallas_call(
        _ffn_body,
        grid_spec=grid_spec,
        out_shape=jax.ShapeDtypeStruct((P, D), jnp.float32),
    )(block_expert, xp, W1, W2, wp)


def kernel(x, W_router, W1, W2):
    B, T, D = x.shape
    N = B * T
    xf = x.reshape(N, D)

    i1, i2, w1, w2, r1, r2, counts = _router(xf, W_router)
    counts = counts.reshape(2 * E)

    P = N * TOP_K + E * BLK
    nb = P // BLK
    rev, wp, dstA, dstB = _sc_finalize(
        i1.reshape(N), i2.reshape(N), r1.reshape(N), r2.reshape(N),
        w1.reshape(N), w2.reshape(N), counts, P)

    padded = ((counts + BLK - 1) // BLK) * BLK
    ends = jnp.cumsum(padded).astype(jnp.int32)
    bstart = jnp.arange(nb, dtype=jnp.int32) * BLK
    block_expert = jnp.minimum(
        jnp.sum((bstart[:, None] >= ends[None, :]).astype(jnp.int32), axis=1),
        E - 1).astype(jnp.int32)

    xp = _sc_gather(xf, rev, rows_per_w=P // NW, chunk=64)
    op = _grouped_ffn(block_expert, xp, W1.astype(jnp.bfloat16),
                      W2.astype(jnp.bfloat16), wp.reshape(P, 1))
    out = _sc_combine(op, dstA, dstB,
                      rows_per_w=N // NW, chunk=32)
    return out.reshape(B, T, D)


# speedup vs baseline: 1.6869x; 1.3735x over previous
import functools

import jax
import jax.numpy as jnp
from jax import lax
from jax.experimental import pallas as pl
from jax.experimental.pallas import tpu as pltpu
from jax.experimental.pallas import tpu_sc as plsc

E = 8
TOP_K = 2
BLK = 256
NC, NS = 2, 16
NW = NC * NS


def _router_body(x_ref, wrt_ref, i1_ref, i2_ref, w1_ref, w2_ref,
                 r1_ref, r2_ref, cnt_ref, tri_ref, carry_ref):
    TB = x_ref.shape[0]
    pid = pl.program_id(0)

    @pl.when(pid == 0)
    def _init():
        r = lax.broadcasted_iota(jnp.int32, (TB, TB), 0)
        c = lax.broadcasted_iota(jnp.int32, (TB, TB), 1)
        tri_ref[...] = jnp.where(r > c, 1.0, 0.0)
        carry_ref[...] = jnp.zeros_like(carry_ref)

    x = x_ref[...]
    logits = jnp.dot(x, wrt_ref[...], preferred_element_type=jnp.float32)
    iota = lax.broadcasted_iota(jnp.int32, logits.shape, 1)
    m1 = jnp.max(logits, axis=1, keepdims=True)
    i1 = jnp.min(jnp.where(logits == m1, iota, E), axis=1, keepdims=True)
    l2 = jnp.where(iota == i1, jnp.float32(-3.0e38), logits)
    m2 = jnp.max(l2, axis=1, keepdims=True)
    i2 = jnp.min(jnp.where(l2 == m2, iota, E), axis=1, keepdims=True)
    e2 = jnp.exp(m2 - m1)
    den = 1.0 + e2
    i1_ref[...] = i1
    i2_ref[...] = i2
    w1_ref[...] = 1.0 / den
    w2_ref[...] = e2 / den

    ioe = lax.broadcasted_iota(jnp.int32, (TB, 2 * E), 1)
    oh1 = jnp.where(ioe == i1, 1.0, 0.0)
    oh2 = jnp.where(ioe == i2, 1.0, 0.0)
    tot = oh1 + oh2
    excl = jnp.dot(tri_ref[...], tot, preferred_element_type=jnp.float32)
    c1 = excl + carry_ref[...]
    r1_ref[...] = jnp.sum(c1 * oh1, axis=1, keepdims=True).astype(jnp.int32)
    r2_ref[...] = jnp.sum(c1 * oh2, axis=1, keepdims=True).astype(jnp.int32)
    new_carry = carry_ref[...] + jnp.sum(tot, axis=0, keepdims=True)
    carry_ref[...] = new_carry
    cnt_ref[...] = new_carry.astype(jnp.int32)


def _router(xf, W_router):
    N, D = xf.shape
    TB = 1024
    grid = (N // TB,)
    out_shapes = (
        jax.ShapeDtypeStruct((N, 1), jnp.int32),
        jax.ShapeDtypeStruct((N, 1), jnp.int32),
        jax.ShapeDtypeStruct((N, 1), jnp.float32),
        jax.ShapeDtypeStruct((N, 1), jnp.float32),
        jax.ShapeDtypeStruct((N, 1), jnp.int32),
        jax.ShapeDtypeStruct((N, 1), jnp.int32),
        jax.ShapeDtypeStruct((1, 2 * E), jnp.int32),
    )
    spec1 = pl.BlockSpec((TB, 1), lambda i: (i, 0))
    cspec = pl.BlockSpec((1, 2 * E), lambda i: (0, 0))
    return pl.pallas_call(
        _router_body,
        grid=grid,
        in_specs=[
            pl.BlockSpec((TB, D), lambda i: (i, 0)),
            pl.BlockSpec((D, E), lambda i: (0, 0)),
        ],
        out_specs=(spec1, spec1, spec1, spec1, spec1, spec1, cspec),
        out_shape=out_shapes,
        scratch_shapes=[
            pltpu.VMEM((TB, TB), jnp.float32),
            pltpu.VMEM((1, 2 * E), jnp.float32),
        ],
    )(xf, W_router.T)


def _sc_finalize(i1, i2, r1, r2, w1, w2, counts16, P):
    N = i1.shape[0]
    ng = N // 16
    mesh = plsc.VectorSubcoreMesh(core_axis_name="c", subcore_axis_name="s",
                                  num_cores=NC, num_subcores=NS)

    @functools.partial(
        pl.kernel,
        out_type=(
            jax.ShapeDtypeStruct((P,), jnp.float32),
            jax.ShapeDtypeStruct((N,), jnp.int32),
            jax.ShapeDtypeStruct((N,), jnp.int32),
        ),
        mesh=mesh,
        scratch_types=[
            pltpu.VMEM((N,), jnp.int32),
            pltpu.VMEM((N,), jnp.int32),
            pltpu.VMEM((N,), jnp.int32),
            pltpu.VMEM((N,), jnp.int32),
            pltpu.VMEM((N,), jnp.float32),
            pltpu.VMEM((N,), jnp.float32),
            pltpu.VMEM((16,), jnp.int32),
            pltpu.VMEM((P,), jnp.float32),
            pltpu.VMEM((N,), jnp.int32),
            pltpu.VMEM((N,), jnp.int32),
        ],
        compiler_params=pltpu.CompilerParams(needs_layout_passes=False),
    )
    def k(i1_h, i2_h, r1_h, r2_h, w1_h, w2_h, c_h,
          wp_h, dA_h, dB_h,
          i1_v, i2_v, r1_v, r2_v, w1_v, w2_v, st_v,
          wp_v, dA_v, dB_v):
        wid = lax.axis_index("s") * NC + lax.axis_index("c")

        @pl.when(wid == 0)
        def _work():
            pltpu.sync_copy(i1_h, i1_v)
            pltpu.sync_copy(i2_h, i2_v)
            pltpu.sync_copy(r1_h, r1_v)
            pltpu.sync_copy(r2_h, r2_v)
            pltpu.sync_copy(w1_h, w1_v)
            pltpu.sync_copy(w2_h, w2_v)
            pltpu.sync_copy(c_h, st_v)
            c16 = st_v[...]
            padded = ((c16 + 255) >> 8) << 8
            cum = plsc.cumsum(padded)
            st_v[...] = cum - padded

            zf16 = jnp.zeros((16,), jnp.float32)

            def zinit(g, carry):
                wp_v[pl.ds(g * 16, 16)] = zf16
                return carry

            lax.fori_loop(0, P // 16, zinit, 0)

            def body(g, carry):
                sl = pl.ds(g * 16, 16)
                e1 = i1_v[sl]
                e2 = i2_v[sl]
                dA = plsc.load_gather(st_v, [e1]) + r1_v[sl]
                dB = plsc.load_gather(st_v, [e2]) + r2_v[sl]
                dA_v[sl] = dA
                dB_v[sl] = dB
                plsc.store_scatter(wp_v, [dA], w1_v[sl])
                plsc.store_scatter(wp_v, [dB], w2_v[sl])
                return carry

            lax.fori_loop(0, ng, body, 0)
            pltpu.sync_copy(wp_v, wp_h)
            pltpu.sync_copy(dA_v, dA_h)
            pltpu.sync_copy(dB_v, dB_h)

    return k(i1, i2, r1, r2, w1, w2, counts16)


def _sc_permute(xf, dstA, dstB, P, chunk):
    N, D = xf.shape
    rows_per_w = N // NW
    nch = rows_per_w // chunk
    mesh = plsc.VectorSubcoreMesh(core_axis_name="c", subcore_axis_name="s",
                                  num_cores=NC, num_subcores=NS)

    @functools.partial(
        pl.kernel,
        out_type=jax.ShapeDtypeStruct((P, D), jnp.float32),
        mesh=mesh,
        scratch_types=[
            pltpu.VMEM((chunk, D), jnp.float32),
            pltpu.VMEM((chunk, D), jnp.float32),
            pltpu.VMEM((chunk,), jnp.int32),
            pltpu.VMEM((chunk,), jnp.int32),
            pltpu.VMEM((chunk,), jnp.int32),
            pltpu.VMEM((chunk,), jnp.int32),
            pltpu.SemaphoreType.DMA,
            pltpu.SemaphoreType.DMA,
            pltpu.SemaphoreType.DMA,
            pltpu.SemaphoreType.DMA,
        ],
    )
    def k(xf_hbm, dA_hbm, dB_hbm, out_hbm,
          s0_v, s1_v, ia0_v, ib0_v, ia1_v, ib1_v, sa0, sb0, sa1, sb1):
        wid = lax.axis_index("s") * NC + lax.axis_index("c")
        base = wid * rows_per_w
        srcs, ias, ibs = (s0_v, s1_v), (ia0_v, ia1_v), (ib0_v, ib1_v)
        sas, sbs = (sa0, sa1), (sb0, sb1)

        def start(j):
            p = j % 2
            b0 = base + j * chunk
            pltpu.sync_copy(xf_hbm.at[pl.ds(b0, chunk)], srcs[p])
            pltpu.sync_copy(dA_hbm.at[pl.ds(b0, chunk)], ias[p])
            pltpu.sync_copy(dB_hbm.at[pl.ds(b0, chunk)], ibs[p])
            ca = pltpu.async_copy(srcs[p], out_hbm.at[ias[p]], sas[p])
            cb = pltpu.async_copy(srcs[p], out_hbm.at[ibs[p]], sbs[p])
            return ca, cb

        cps = [None, None]
        cps[0] = start(0)
        for j in range(nch):
            p = j % 2
            if j + 1 < nch:
                cps[(j + 1) % 2] = start(j + 1)
            cps[p][0].wait()
            cps[p][1].wait()

    return k(xf, dstA, dstB)


def _sc_combine(table, idx_a, idx_b, rows_per_w, chunk):
    N = idx_a.shape[0]
    D = table.shape[1]
    nch = rows_per_w // chunk
    nvec = D // 16
    mesh = plsc.VectorSubcoreMesh(core_axis_name="c", subcore_axis_name="s",
                                  num_cores=NC, num_subcores=NS)

    @functools.partial(
        pl.kernel,
        out_type=jax.ShapeDtypeStruct((N, D), jnp.float32),
        mesh=mesh,
        scratch_types=[
            pltpu.VMEM((chunk,), jnp.int32),
            pltpu.VMEM((chunk,), jnp.int32),
            pltpu.VMEM((chunk, D), jnp.float32),
            pltpu.VMEM((chunk, D), jnp.float32),
            pltpu.SemaphoreType.DMA,
            pltpu.SemaphoreType.DMA,
        ],
    )
    def k(table_hbm, ia_hbm, ib_hbm, out_hbm, ia_v, ib_v, a_v, b_v, sa, sb):
        wid = lax.axis_index("s") * NC + lax.axis_index("c")
        base = wid * rows_per_w

        def body(j, carry):
            b0 = base + j * chunk
            pltpu.sync_copy(ia_hbm.at[pl.ds(b0, chunk)], ia_v)
            pltpu.sync_copy(ib_hbm.at[pl.ds(b0, chunk)], ib_v)
            ca = pltpu.async_copy(table_hbm.at[ia_v], a_v, sa)
            cb = pltpu.async_copy(table_hbm.at[ib_v], b_v, sb)
            ca.wait()
            cb.wait()

            def row(r, carry2):
                for v in range(nvec):
                    sl = pl.ds(v * 16, 16)
                    a_v[r, sl] = a_v[r, sl] + b_v[r, sl]
                return carry2

            lax.fori_loop(0, chunk, row, 0)
            pltpu.sync_copy(a_v, out_hbm.at[pl.ds(b0, chunk)])
            return carry

        lax.fori_loop(0, nch, body, 0)

    return k(table, idx_a, idx_b)


def _ffn_body(be_ref, xp_ref, w1_ref, w2_ref, wp_ref, out_ref):
    e = be_ref[pl.program_id(0)]
    x = xp_ref[...].astype(jnp.bfloat16)
    h = jnp.dot(x, w1_ref[e], preferred_element_type=jnp.float32)
    h = 0.5 * h * (1.0 + lax.erf(h * 0.7071067811865476))
    o = jnp.dot(h.astype(jnp.bfloat16), w2_ref[e],
                preferred_element_type=jnp.float32)
    out_ref[...] = o * wp_ref[...]


def _grouped_ffn(block_expert, xp, W1, W2, wp):
    P, D = xp.shape
    FF = W1.shape[2]
    nb = P // BLK
    grid_spec = pltpu.PrefetchScalarGridSpec(
        num_scalar_prefetch=1,
        grid=(nb,),
        in_specs=[
            pl.BlockSpec((BLK, D), lambda i, be: (i, 0)),
            pl.BlockSpec((E, D, FF), lambda i, be: (0, 0, 0)),
            pl.BlockSpec((E, FF, D), lambda i, be: (0, 0, 0)),
            pl.BlockSpec((BLK, 1), lambda i, be: (i, 0)),
        ],
        out_specs=pl.BlockSpec((BLK, D), lambda i, be: (i, 0)),
    )
    return pl.pallas_call(
        _ffn_body,
        grid_spec=grid_spec,
        out_shape=jax.ShapeDtypeStruct((P, D), jnp.float32),
    )(block_expert, xp, W1, W2, wp)


def kernel(x, W_router, W1, W2):
    B, T, D = x.shape
    N = B * T
    xf = x.reshape(N, D)

    i1, i2, w1, w2, r1, r2, counts = _router(xf, W_router)
    counts = counts.reshape(2 * E)

    P = N * TOP_K + E * BLK
    nb = P // BLK
    wp, dstA, dstB = _sc_finalize(
        i1.reshape(N), i2.reshape(N), r1.reshape(N), r2.reshape(N),
        w1.reshape(N), w2.reshape(N), counts, P)

    padded = ((counts + BLK - 1) // BLK) * BLK
    ends = jnp.cumsum(padded).astype(jnp.int32)
    bstart = jnp.arange(nb, dtype=jnp.int32) * BLK
    block_expert = jnp.minimum(
        jnp.sum((bstart[:, None] >= ends[None, :]).astype(jnp.int32), axis=1),
        E - 1).astype(jnp.int32)

    xp = _sc_permute(xf, dstA, dstB, P, chunk=64)
    op = _grouped_ffn(block_expert, xp, W1.astype(jnp.bfloat16),
                      W2.astype(jnp.bfloat16), wp.reshape(P, 1))
    out = _sc_combine(op, dstA, dstB,
                      rows_per_w=N // NW, chunk=32)
    return out.reshape(B, T, D)


# speedup vs baseline: 1.7698x; 1.0491x over previous
import functools

import jax
import jax.numpy as jnp
from jax import lax
from jax.experimental import pallas as pl
from jax.experimental.pallas import tpu as pltpu
from jax.experimental.pallas import tpu_sc as plsc

E = 8
TOP_K = 2
BLK = 256
NC, NS = 2, 16
NW = NC * NS


def _router_body(x_ref, wrt_ref, i1_ref, i2_ref, w1_ref, w2_ref,
                 r1_ref, r2_ref, cnt_ref, tri_ref, carry_ref):
    TB = x_ref.shape[0]
    pid = pl.program_id(0)

    @pl.when(pid == 0)
    def _init():
        r = lax.broadcasted_iota(jnp.int32, (TB, TB), 0)
        c = lax.broadcasted_iota(jnp.int32, (TB, TB), 1)
        tri_ref[...] = jnp.where(r > c, 1.0, 0.0)
        carry_ref[...] = jnp.zeros_like(carry_ref)

    x = x_ref[...]
    logits = jnp.dot(x, wrt_ref[...], preferred_element_type=jnp.float32)
    iota = lax.broadcasted_iota(jnp.int32, logits.shape, 1)
    m1 = jnp.max(logits, axis=1, keepdims=True)
    i1 = jnp.min(jnp.where(logits == m1, iota, E), axis=1, keepdims=True)
    l2 = jnp.where(iota == i1, jnp.float32(-3.0e38), logits)
    m2 = jnp.max(l2, axis=1, keepdims=True)
    i2 = jnp.min(jnp.where(l2 == m2, iota, E), axis=1, keepdims=True)
    e2 = jnp.exp(m2 - m1)
    den = 1.0 + e2
    i1_ref[...] = i1
    i2_ref[...] = i2
    w1_ref[...] = 1.0 / den
    w2_ref[...] = e2 / den

    ioe = lax.broadcasted_iota(jnp.int32, (TB, 2 * E), 1)
    oh1 = jnp.where(ioe == i1, 1.0, 0.0)
    oh2 = jnp.where(ioe == i2, 1.0, 0.0)
    tot = oh1 + oh2
    excl = jnp.dot(tri_ref[...], tot, preferred_element_type=jnp.float32)
    c1 = excl + carry_ref[...]
    r1_ref[...] = jnp.sum(c1 * oh1, axis=1, keepdims=True).astype(jnp.int32)
    r2_ref[...] = jnp.sum(c1 * oh2, axis=1, keepdims=True).astype(jnp.int32)
    new_carry = carry_ref[...] + jnp.sum(tot, axis=0, keepdims=True)
    carry_ref[...] = new_carry
    cnt_ref[...] = new_carry.astype(jnp.int32)


def _router(xf, W_router):
    N, D = xf.shape
    TB = 1024
    grid = (N // TB,)
    out_shapes = (
        jax.ShapeDtypeStruct((N, 1), jnp.int32),
        jax.ShapeDtypeStruct((N, 1), jnp.int32),
        jax.ShapeDtypeStruct((N, 1), jnp.float32),
        jax.ShapeDtypeStruct((N, 1), jnp.float32),
        jax.ShapeDtypeStruct((N, 1), jnp.int32),
        jax.ShapeDtypeStruct((N, 1), jnp.int32),
        jax.ShapeDtypeStruct((1, 2 * E), jnp.int32),
    )
    spec1 = pl.BlockSpec((TB, 1), lambda i: (i, 0))
    cspec = pl.BlockSpec((1, 2 * E), lambda i: (0, 0))
    return pl.pallas_call(
        _router_body,
        grid=grid,
        in_specs=[
            pl.BlockSpec((TB, D), lambda i: (i, 0)),
            pl.BlockSpec((D, E), lambda i: (0, 0)),
        ],
        out_specs=(spec1, spec1, spec1, spec1, spec1, spec1, cspec),
        out_shape=out_shapes,
        scratch_shapes=[
            pltpu.VMEM((TB, TB), jnp.float32),
            pltpu.VMEM((1, 2 * E), jnp.float32),
        ],
    )(xf, W_router.T)


def _sc_finalize(i1, i2, r1, r2, w1, w2, counts16, P):
    N = i1.shape[0]
    ng = N // 16
    mesh = plsc.VectorSubcoreMesh(core_axis_name="c", subcore_axis_name="s",
                                  num_cores=NC, num_subcores=NS)

    @functools.partial(
        pl.kernel,
        out_type=(
            jax.ShapeDtypeStruct((P,), jnp.float32),
            jax.ShapeDtypeStruct((N,), jnp.int32),
            jax.ShapeDtypeStruct((N,), jnp.int32),
        ),
        mesh=mesh,
        scratch_types=[
            pltpu.VMEM((N,), jnp.int32),
            pltpu.VMEM((N,), jnp.int32),
            pltpu.VMEM((N,), jnp.int32),
            pltpu.VMEM((N,), jnp.int32),
            pltpu.VMEM((N,), jnp.float32),
            pltpu.VMEM((N,), jnp.float32),
            pltpu.VMEM((16,), jnp.int32),
            pltpu.VMEM((P,), jnp.float32),
            pltpu.VMEM((N,), jnp.int32),
            pltpu.VMEM((N,), jnp.int32),
        ],
        compiler_params=pltpu.CompilerParams(needs_layout_passes=False),
    )
    def k(i1_h, i2_h, r1_h, r2_h, w1_h, w2_h, c_h,
          wp_h, dA_h, dB_h,
          i1_v, i2_v, r1_v, r2_v, w1_v, w2_v, st_v,
          wp_v, dA_v, dB_v):
        wid = lax.axis_index("s") * NC + lax.axis_index("c")

        @pl.when(wid == 0)
        def _work():
            pltpu.sync_copy(i1_h, i1_v)
            pltpu.sync_copy(i2_h, i2_v)
            pltpu.sync_copy(r1_h, r1_v)
            pltpu.sync_copy(r2_h, r2_v)
            pltpu.sync_copy(w1_h, w1_v)
            pltpu.sync_copy(w2_h, w2_v)
            pltpu.sync_copy(c_h, st_v)
            c16 = st_v[...]
            padded = ((c16 + 255) >> 8) << 8
            cum = plsc.cumsum(padded)
            st_v[...] = cum - padded

            zf16 = jnp.zeros((16,), jnp.float32)

            def zinit(g, carry):
                wp_v[pl.ds(g * 16, 16)] = zf16
                return carry

            lax.fori_loop(0, P // 16, zinit, 0)

            def body(g, carry):
                sl = pl.ds(g * 16, 16)
                e1 = i1_v[sl]
                e2 = i2_v[sl]
                dA = plsc.load_gather(st_v, [e1]) + r1_v[sl]
                dB = plsc.load_gather(st_v, [e2]) + r2_v[sl]
                dA_v[sl] = dA
                dB_v[sl] = dB
                plsc.store_scatter(wp_v, [dA], w1_v[sl])
                plsc.store_scatter(wp_v, [dB], w2_v[sl])
                return carry

            lax.fori_loop(0, ng, body, 0)
            pltpu.sync_copy(wp_v, wp_h)
            pltpu.sync_copy(dA_v, dA_h)
            pltpu.sync_copy(dB_v, dB_h)

    return k(i1, i2, r1, r2, w1, w2, counts16)


def _sc_permute(xf, dstA, dstB, P, chunk):
    N, D = xf.shape
    rows_per_w = N // NW
    nch = rows_per_w // chunk
    mesh = plsc.VectorSubcoreMesh(core_axis_name="c", subcore_axis_name="s",
                                  num_cores=NC, num_subcores=NS)

    @functools.partial(
        pl.kernel,
        out_type=jax.ShapeDtypeStruct((P, D), jnp.float32),
        mesh=mesh,
        scratch_types=[
            pltpu.VMEM((chunk, D), jnp.float32),
            pltpu.VMEM((chunk, D), jnp.float32),
            pltpu.VMEM((chunk,), jnp.int32),
            pltpu.VMEM((chunk,), jnp.int32),
            pltpu.VMEM((chunk,), jnp.int32),
            pltpu.VMEM((chunk,), jnp.int32),
            pltpu.SemaphoreType.DMA,
            pltpu.SemaphoreType.DMA,
            pltpu.SemaphoreType.DMA,
            pltpu.SemaphoreType.DMA,
        ],
    )
    def k(xf_hbm, dA_hbm, dB_hbm, out_hbm,
          s0_v, s1_v, ia0_v, ib0_v, ia1_v, ib1_v, sa0, sb0, sa1, sb1):
        wid = lax.axis_index("s") * NC + lax.axis_index("c")
        base = wid * rows_per_w
        srcs, ias, ibs = (s0_v, s1_v), (ia0_v, ia1_v), (ib0_v, ib1_v)
        sas, sbs = (sa0, sa1), (sb0, sb1)

        def start(j):
            p = j % 2
            b0 = base + j * chunk
            pltpu.sync_copy(xf_hbm.at[pl.ds(b0, chunk)], srcs[p])
            pltpu.sync_copy(dA_hbm.at[pl.ds(b0, chunk)], ias[p])
            pltpu.sync_copy(dB_hbm.at[pl.ds(b0, chunk)], ibs[p])
            ca = pltpu.async_copy(srcs[p], out_hbm.at[ias[p]], sas[p])
            cb = pltpu.async_copy(srcs[p], out_hbm.at[ibs[p]], sbs[p])
            return ca, cb

        cps = [None, None]
        cps[0] = start(0)
        for j in range(nch):
            p = j % 2
            if j + 1 < nch:
                cps[(j + 1) % 2] = start(j + 1)
            cps[p][0].wait()
            cps[p][1].wait()

    return k(xf, dstA, dstB)


def _sc_combine(table, idx_a, idx_b, rows_per_w, chunk):
    N = idx_a.shape[0]
    D = table.shape[1]
    nch = rows_per_w // chunk
    nvec = D // 16
    mesh = plsc.VectorSubcoreMesh(core_axis_name="c", subcore_axis_name="s",
                                  num_cores=NC, num_subcores=NS)

    @functools.partial(
        pl.kernel,
        out_type=jax.ShapeDtypeStruct((N, D), jnp.float32),
        mesh=mesh,
        scratch_types=[
            pltpu.VMEM((chunk,), jnp.int32),
            pltpu.VMEM((chunk,), jnp.int32),
            pltpu.VMEM((chunk,), jnp.int32),
            pltpu.VMEM((chunk,), jnp.int32),
            pltpu.VMEM((chunk, D), jnp.float32),
            pltpu.VMEM((chunk, D), jnp.float32),
            pltpu.VMEM((chunk, D), jnp.float32),
            pltpu.VMEM((chunk, D), jnp.float32),
            pltpu.SemaphoreType.DMA,
            pltpu.SemaphoreType.DMA,
            pltpu.SemaphoreType.DMA,
            pltpu.SemaphoreType.DMA,
        ],
    )
    def k(table_hbm, ia_hbm, ib_hbm, out_hbm,
          ia0_v, ib0_v, ia1_v, ib1_v, a0_v, b0_v, a1_v, b1_v,
          sa0, sb0, sa1, sb1):
        wid = lax.axis_index("s") * NC + lax.axis_index("c")
        base = wid * rows_per_w
        ias, ibs = (ia0_v, ia1_v), (ib0_v, ib1_v)
        avs, bvs = (a0_v, a1_v), (b0_v, b1_v)
        sas, sbs = (sa0, sa1), (sb0, sb1)

        def start(j):
            p = j % 2
            b0 = base + j * chunk
            pltpu.sync_copy(ia_hbm.at[pl.ds(b0, chunk)], ias[p])
            pltpu.sync_copy(ib_hbm.at[pl.ds(b0, chunk)], ibs[p])
            ca = pltpu.async_copy(table_hbm.at[ias[p]], avs[p], sas[p])
            cb = pltpu.async_copy(table_hbm.at[ibs[p]], bvs[p], sbs[p])
            return ca, cb

        cps = [None, None]
        cps[0] = start(0)
        for j in range(nch):
            p = j % 2
            if j + 1 < nch:
                cps[(j + 1) % 2] = start(j + 1)
            cps[p][0].wait()
            cps[p][1].wait()
            a_v, b_v = avs[p], bvs[p]

            def row(r, carry2):
                for v in range(nvec):
                    sl = pl.ds(v * 16, 16)
                    a_v[r, sl] = a_v[r, sl] + b_v[r, sl]
                return carry2

            lax.fori_loop(0, chunk, row, 0)
            pltpu.sync_copy(a_v, out_hbm.at[pl.ds(base + j * chunk, chunk)])

    return k(table, idx_a, idx_b)


def _ffn_body(be_ref, xp_ref, w1_ref, w2_ref, wp_ref, out_ref):
    e = be_ref[pl.program_id(0)]
    x = xp_ref[...].astype(jnp.bfloat16)
    h = jnp.dot(x, w1_ref[e], preferred_element_type=jnp.float32)
    h = 0.5 * h * (1.0 + lax.erf(h * 0.7071067811865476))
    o = jnp.dot(h.astype(jnp.bfloat16), w2_ref[e],
                preferred_element_type=jnp.float32)
    out_ref[...] = o * wp_ref[...]


def _grouped_ffn(block_expert, xp, W1, W2, wp):
    P, D = xp.shape
    FF = W1.shape[2]
    nb = P // BLK
    grid_spec = pltpu.PrefetchScalarGridSpec(
        num_scalar_prefetch=1,
        grid=(nb,),
        in_specs=[
            pl.BlockSpec((BLK, D), lambda i, be: (i, 0)),
            pl.BlockSpec((E, D, FF), lambda i, be: (0, 0, 0)),
            pl.BlockSpec((E, FF, D), lambda i, be: (0, 0, 0)),
            pl.BlockSpec((BLK, 1), lambda i, be: (i, 0)),
        ],
        out_specs=pl.BlockSpec((BLK, D), lambda i, be: (i, 0)),
    )
    return pl.pallas_call(
        _ffn_body,
        grid_spec=grid_spec,
        out_shape=jax.ShapeDtypeStruct((P, D), jnp.float32),
    )(block_expert, xp, W1, W2, wp)


def kernel(x, W_router, W1, W2):
    B, T, D = x.shape
    N = B * T
    xf = x.reshape(N, D)

    i1, i2, w1, w2, r1, r2, counts = _router(xf, W_router)
    counts = counts.reshape(2 * E)

    P = N * TOP_K + E * BLK
    nb = P // BLK
    wp, dstA, dstB = _sc_finalize(
        i1.reshape(N), i2.reshape(N), r1.reshape(N), r2.reshape(N),
        w1.reshape(N), w2.reshape(N), counts, P)

    padded = ((counts + BLK - 1) // BLK) * BLK
    ends = jnp.cumsum(padded).astype(jnp.int32)
    bstart = jnp.arange(nb, dtype=jnp.int32) * BLK
    block_expert = jnp.minimum(
        jnp.sum((bstart[:, None] >= ends[None, :]).astype(jnp.int32), axis=1),
        E - 1).astype(jnp.int32)

    xp = _sc_permute(xf, dstA, dstB, P, chunk=64)
    op = _grouped_ffn(block_expert, xp, W1.astype(jnp.bfloat16),
                      W2.astype(jnp.bfloat16), wp.reshape(P, 1))
    out = _sc_combine(op, dstA, dstB,
                      rows_per_w=N // NW, chunk=32)
    return out.reshape(B, T, D)
